# baseline (speedup 1.0000x reference)
import functools

import jax
import jax.numpy as jnp
from jax import lax
from jax.experimental import pallas as pl
from jax.experimental.pallas import tpu as pltpu
from jax.experimental.pallas import tpu_sc as plsc

B = 16384
D = 64
PW = 2 * D
L = 16
NC, NS = 2, 16
NW = NC * NS
BPW = B // NW
CH = 128
NCH = BPW // CH
G = 16
NG = BPW // G

_mesh = plsc.VectorSubcoreMesh(core_axis_name="c", subcore_axis_name="s")


def _tec_body(ids_hbm, tab2_hbm, out_hbm, idr_v, pidx_v, par_v, slots_v,
              rows_v, sem, osem):
    wid = lax.axis_index("s") * NC + lax.axis_index("c")
    base = wid * BPW
    iota = lax.iota(jnp.int32, L)

    for ch in range(NCH):
        pltpu.sync_copy(ids_hbm.at[wid * NCH + ch], idr_v.at[ch, 0])
    for ch in range(NCH):
        for k in range(CH // L):
            t16 = idr_v[ch, 0, pl.ds(k * L, L)]
            pidx_v[ch, 0, pl.ds(k * L, L)] = lax.shift_right_logical(t16, 1)
            par_v[pl.ds(ch * CH + k * L, L)] = lax.bitwise_and(
                t16, jnp.int32(1))

    for ch in range(NCH):
        pltpu.async_copy(
            tab2_hbm.at[pidx_v.at[ch, 0]],
            slots_v.at[pl.ds(ch * CH, CH)], sem)
    for ch in range(NCH):
        pltpu.make_async_copy(
            tab2_hbm.at[pidx_v.at[ch, 0]],
            slots_v.at[pl.ds(ch * CH, CH)], sem).wait()

    def group(g, carry):
        r16 = g * G + iota
        par16 = par_v[pl.ds(g * G, L)]
        coff = par16 * jnp.int32(D)
        cols = []
        s = None
        for j in range(D):
            col = plsc.load_gather(slots_v, [r16, coff + jnp.int32(j)])
            cols.append(col)
            sq = col * col
            s = sq if s is None else s + sq
        s = jnp.maximum(s, jnp.float32(1e-24))
        half = s * jnp.float32(0.5)
        yi = jnp.int32(0x5F3759DF) - lax.shift_right_logical(
            plsc.bitcast(s, jnp.int32), 1)
        y = plsc.bitcast(yi, jnp.float32)
        for _ in range(3):
            y = y * (jnp.float32(1.5) - half * y * y)
        obase = (g * G + iota) * jnp.int32(D)
        for j in range(D):
            plsc.store_scatter(rows_v, [obase + jnp.int32(j)], cols[j] * y)
        return carry

    lax.fori_loop(0, NG, group, 0)
    pltpu.async_copy(rows_v, out_hbm.at[pl.ds(base * D, BPW * D)], osem)
    pltpu.make_async_copy(
        rows_v, out_hbm.at[pl.ds(base * D, BPW * D)], osem).wait()


@functools.partial(
    pl.kernel,
    out_type=jax.ShapeDtypeStruct((B * D,), jnp.float32),
    mesh=_mesh,
    compiler_params=pltpu.CompilerParams(
        needs_layout_passes=False, use_tc_tiling_on_sc=True),
    scratch_types=[
        pltpu.VMEM((NCH, 8, CH), jnp.int32),
        pltpu.VMEM((NCH, 8, CH), jnp.int32),
        pltpu.VMEM((BPW,), jnp.int32),
        pltpu.VMEM((BPW, PW), jnp.float32),
        pltpu.VMEM((BPW * D,), jnp.float32),
        pltpu.SemaphoreType.DMA,
        pltpu.SemaphoreType.DMA,
    ],
)
def _embed_norm(ids_hbm, tab2_hbm, out_hbm, idr_v, pidx_v, par_v, slots_v,
                rows_v, sem, osem):
    _tec_body(ids_hbm, tab2_hbm, out_hbm, idr_v, pidx_v, par_v, slots_v,
              rows_v, sem, osem)


def kernel(task_ids, table):
    ids2 = task_ids.astype(jnp.int32).reshape(B // CH, CH)
    tab2 = table.reshape(500000, PW)
    flat = _embed_norm(ids2, tab2)
    return flat.reshape(B, D)
